# speedup vs baseline: 1.0314x; 1.0314x over previous
import functools
import math

import jax
import jax.numpy as jnp
from jax import lax
from jax.experimental import pallas as pl
from jax.experimental.pallas import tpu as pltpu
from jax.experimental.pallas import tpu_sc as plsc


@functools.lru_cache(maxsize=None)
def _make_gather(V, D, B):
    info = plsc.get_sparse_core_info()
    NC, NS, L = info.num_cores, info.num_subcores, info.num_lanes
    NW = NC * NS
    assert D % L == 0 and B % (8 * NW) == 0
    b_per_w = B // NW
    C = 32
    n_chunks = b_per_w // C
    scale = math.sqrt(D)
    mesh = plsc.VectorSubcoreMesh(core_axis_name="c", subcore_axis_name="s")

    @functools.partial(
        pl.kernel,
        mesh=mesh,
        out_type=jax.ShapeDtypeStruct((B, D), jnp.float32),
        scratch_types=[
            pltpu.VMEM((b_per_w,), jnp.int32),
            pltpu.VMEM((C, D), jnp.float32),
            pltpu.SemaphoreType.DMA,
        ],
    )
    def gather_scale(table_hbm, idx_hbm, out_hbm, idx_v, rows_v, sem):
        wid = lax.axis_index("s") * NC + lax.axis_index("c")
        base = wid * b_per_w
        pltpu.sync_copy(idx_hbm.at[pl.ds(base, b_per_w)], idx_v)

        def chunk_body(c, carry):
            pltpu.async_copy(
                table_hbm.at[idx_v.at[pl.ds(c * C, C)]], rows_v, sem
            ).wait()

            def row_body(i, carry2):
                for j in range(D // L):
                    rows_v[i, pl.ds(j * L, L)] = (
                        rows_v[i, pl.ds(j * L, L)] * scale
                    )
                return carry2

            lax.fori_loop(0, C, row_body, 0)
            pltpu.sync_copy(rows_v, out_hbm.at[pl.ds(base + c * C, C)])
            return carry

        lax.fori_loop(0, n_chunks, chunk_body, 0)

    return gather_scale


def kernel(x, emb):
    V, D = emb.shape
    B = x.size
    x_flat = x.reshape(B).astype(jnp.int32)
    out = _make_gather(V, D, B)(emb, x_flat)
    return out.reshape(x.shape + (D,))


# speedup vs baseline: 1.2953x; 1.2559x over previous
import functools
import math

import jax
import jax.numpy as jnp
from jax import lax
from jax.experimental import pallas as pl
from jax.experimental.pallas import tpu as pltpu
from jax.experimental.pallas import tpu_sc as plsc


@functools.lru_cache(maxsize=None)
def _make_gather(V, D, B):
    info = plsc.get_sparse_core_info()
    NC, NS, L = info.num_cores, info.num_subcores, info.num_lanes
    NW = NC * NS
    assert D % L == 0 and B % (8 * NW) == 0
    b_per_w = B // NW
    C = 32
    n_chunks = b_per_w // C
    scale = math.sqrt(D)
    mesh = plsc.VectorSubcoreMesh(core_axis_name="c", subcore_axis_name="s")

    nbuf = 2

    @functools.partial(
        pl.kernel,
        mesh=mesh,
        out_type=jax.ShapeDtypeStruct((B, D), jnp.float32),
        scratch_types=[
            pltpu.VMEM((b_per_w,), jnp.int32),
        ]
        + [pltpu.VMEM((C, D), jnp.float32) for _ in range(nbuf)]
        + [pltpu.SemaphoreType.DMA for _ in range(2 * nbuf)],
    )
    def gather_scale(table_hbm, idx_hbm, out_hbm, idx_v, *bufs_and_sems):
        rows = bufs_and_sems[:nbuf]
        gsem = bufs_and_sems[nbuf : 2 * nbuf]
        ssem = bufs_and_sems[2 * nbuf : 3 * nbuf]
        wid = lax.axis_index("s") * NC + lax.axis_index("c")
        base = wid * b_per_w
        pltpu.sync_copy(idx_hbm.at[pl.ds(base, b_per_w)], idx_v)

        def scale_buf(buf):
            def row_body(i, carry):
                for j in range(D // L):
                    buf[i, pl.ds(j * L, L)] = buf[i, pl.ds(j * L, L)] * scale
                return carry

            lax.fori_loop(0, C, row_body, 0)

        gather = [None] * n_chunks
        store = [None] * n_chunks
        gather[0] = pltpu.async_copy(
            table_hbm.at[idx_v.at[pl.ds(0, C)]], rows[0], gsem[0]
        )
        for c in range(n_chunks):
            p = c % nbuf
            if c + 1 < n_chunks:
                q = (c + 1) % nbuf
                if c - 1 >= 0:
                    store[c - 1].wait()
                gather[c + 1] = pltpu.async_copy(
                    table_hbm.at[idx_v.at[pl.ds((c + 1) * C, C)]],
                    rows[q],
                    gsem[q],
                )
            gather[c].wait()
            scale_buf(rows[p])
            store[c] = pltpu.async_copy(
                rows[p], out_hbm.at[pl.ds(base + c * C, C)], ssem[p]
            )
        for c in range(max(0, n_chunks - nbuf), n_chunks):
            store[c].wait()

    return gather_scale


def kernel(x, emb):
    V, D = emb.shape
    B = x.size
    x_flat = x.reshape(B).astype(jnp.int32)
    out = _make_gather(V, D, B)(emb, x_flat)
    return out.reshape(x.shape + (D,))
